# baseline (speedup 1.0000x reference)
import functools

import jax
import jax.numpy as jnp
from jax import lax
from jax.experimental import pallas as pl
from jax.experimental.pallas import tpu as pltpu
from jax.experimental.pallas import tpu_sc as plsc

_N_ROWS = 16384
_N_ATOMS = 200
_NC = 2
_NS = 16
_NW = _NC * _NS
_ROWS_PER_W = _N_ROWS // _NW
_GROUPS = _ROWS_PER_W // 16


def _sc_body(species_hbm, energies_hbm, se_hbm, out_hbm,
             spec_v, en_v, out_v, table_v):
  cid = lax.axis_index("c")
  sid = lax.axis_index("s")
  wid = sid * _NC + cid
  row0 = wid * _ROWS_PER_W

  pltpu.sync_copy(se_hbm, table_v)
  pltpu.sync_copy(
      species_hbm.at[pl.ds(row0 * _N_ATOMS, _ROWS_PER_W * _N_ATOMS)], spec_v)
  pltpu.sync_copy(energies_hbm.at[pl.ds(row0, _ROWS_PER_W)], en_v)

  lanes = lax.iota(jnp.int32, 16)

  def group_body(g, carry):
    rbase = (g * 16 + lanes) * _N_ATOMS

    def jbody(j, state):
      acc, idx = state
      sv = plsc.load_gather(spec_v, [idx])
      ev = plsc.load_gather(table_v, [sv])
      return acc + ev, idx + 1

    acc, _ = lax.fori_loop(0, _N_ATOMS, jbody,
                           (jnp.zeros((16,), jnp.float32), rbase),
                           unroll=8)
    off = pl.multiple_of(g * 16, 16)
    out_v[pl.ds(off, 16)] = acc + en_v[pl.ds(off, 16)]
    return carry

  lax.fori_loop(0, _GROUPS, group_body, 0)
  pltpu.sync_copy(out_v, out_hbm.at[pl.ds(row0, _ROWS_PER_W)])


@jax.jit
def _shifted(species, energies, self_energies):
  mesh = plsc.VectorSubcoreMesh(core_axis_name="c", subcore_axis_name="s")
  fn = pl.kernel(
      _sc_body,
      out_type=jax.ShapeDtypeStruct((_N_ROWS,), jnp.float32),
      mesh=mesh,
      compiler_params=pltpu.CompilerParams(needs_layout_passes=False),
      scratch_types=[
          pltpu.VMEM((_ROWS_PER_W * _N_ATOMS,), jnp.int32),
          pltpu.VMEM((_ROWS_PER_W,), jnp.float32),
          pltpu.VMEM((_ROWS_PER_W,), jnp.float32),
          pltpu.VMEM((10,), jnp.float32),
      ],
  )
  return fn(species.reshape(-1), energies, self_energies)


def kernel(species, energies, self_energies):
  out = _shifted(species.astype(jnp.int32), energies, self_energies)
  return (species, out.astype(energies.dtype))
